# baseline (speedup 1.0000x reference)
import jax
import jax.numpy as jnp
from jax.experimental import pallas as pl

N = 50000
H = 128
NODE_TYPES = ["Policy", "Control", "ComplianceRequirement", "Risk"]
ETYPES = [
    ("Policy", "governs", "Control"),
    ("Control", "governed_by", "Policy"),
    ("ComplianceRequirement", "requires", "Control"),
    ("Control", "satisfies", "ComplianceRequirement"),
    ("Control", "mitigates", "Risk"),
    ("Risk", "mitigated_by", "Control"),
]


def _relu_kernel(x_ref, o_ref):
    o_ref[...] = jnp.maximum(x_ref[...], 0.0)


def _relu4(x):
    BN = 1000
    return pl.pallas_call(
        _relu_kernel,
        grid=(4, N // BN),
        in_specs=[pl.BlockSpec((1, BN, H), lambda t, i: (t, i, 0))],
        out_specs=pl.BlockSpec((1, BN, H), lambda t, i: (t, i, 0)),
        out_shape=jax.ShapeDtypeStruct((4, N, H), jnp.float32),
    )(x)


def _graphconv(x_src, W, b, src, dst):
    deg_out = jnp.clip(jnp.zeros((N,), jnp.float32).at[src].add(1.0), 1.0, None)
    deg_in = jnp.clip(jnp.zeros((N,), jnp.float32).at[dst].add(1.0), 1.0, None)
    feat = (x_src * (deg_out ** -0.5)[:, None]) @ W
    agg = jax.ops.segment_sum(feat[src], dst, num_segments=N)
    return agg * (deg_in ** -0.5)[:, None] + b


def _hetero(h, prefix, d):
    out = {nt: jnp.zeros((N, H), jnp.float32) for nt in NODE_TYPES}
    for (st, en, dt) in ETYPES:
        ei = d[f"edge_{en}"]
        out[dt] = out[dt] + _graphconv(h[st], d[f"{prefix}_W_{en}"], d[f"{prefix}_b_{en}"], ei[0], ei[1])
    return out


def kernel(feat_Policy, Win_Policy, bin_Policy, feat_Control, Win_Control, bin_Control, feat_ComplianceRequirement, Win_ComplianceRequirement, bin_ComplianceRequirement, feat_Risk, Win_Risk, bin_Risk, edge_governs, l1_W_governs, l1_b_governs, l2_W_governs, l2_b_governs, edge_governed_by, l1_W_governed_by, l1_b_governed_by, l2_W_governed_by, l2_b_governed_by, edge_requires, l1_W_requires, l1_b_requires, l2_W_requires, l2_b_requires, edge_satisfies, l1_W_satisfies, l1_b_satisfies, l2_W_satisfies, l2_b_satisfies, edge_mitigates, l1_W_mitigates, l1_b_mitigates, l2_W_mitigates, l2_b_mitigates, edge_mitigated_by, l1_W_mitigated_by, l1_b_mitigated_by, l2_W_mitigated_by, l2_b_mitigated_by):
    d = dict(locals())
    h = {nt: jax.nn.relu(d[f"feat_{nt}"] @ d[f"Win_{nt}"] + d[f"bin_{nt}"]) for nt in NODE_TYPES}
    h1 = _hetero(h, "l1", d)
    h1 = {k: jax.nn.relu(v) for k, v in h1.items()}
    h2 = _hetero(h1, "l2", d)
    return _relu4(jnp.stack([h2[nt] for nt in NODE_TYPES], axis=0))


# speedup vs baseline: 3.3914x; 3.3914x over previous
import functools

import jax
import jax.numpy as jnp
from jax import lax
from jax.experimental import pallas as pl
from jax.experimental.pallas import tpu as pltpu
from jax.experimental.pallas import tpu_sc as plsc

N = 50000
H = 128
E = 625000
F_IN = 385
CW = 32
NCHUNK = 4
EB = 128
NBLK = 306
EP = 16 * NBLK * EB
NJ = N + 48
SPAN = NJ // 16
WSPAN_LAST = N - 15 * SPAN
BN = 1000
GRID = N // BN

EN_NAMES = ["governs", "governed_by", "requires", "satisfies", "mitigates", "mitigated_by"]
SRC_OF_EN = [0, 1, 2, 1, 1, 3]
DST_LISTS = [[1], [0, 2, 5], [3], [4]]


def _proj_body(x_ref, w_ref, b_ref, o_ref):
    y = jnp.dot(x_ref[...], w_ref[...], preferred_element_type=jnp.float32)
    o_ref[...] = jnp.maximum(y + b_ref[...], 0.0)


def _proj(x, w, b):
    return pl.pallas_call(
        _proj_body,
        grid=(GRID,),
        in_specs=[
            pl.BlockSpec((BN, F_IN), lambda i: (i, 0)),
            pl.BlockSpec((F_IN, H), lambda i: (0, 0)),
            pl.BlockSpec((1, H), lambda i: (0, 0)),
        ],
        out_specs=pl.BlockSpec((BN, H), lambda i: (i, 0)),
        out_shape=jax.ShapeDtypeStruct((N, H), jnp.float32),
    )(x, w, b.reshape(1, H))


def _ymm_body(h_ref, w_ref, deg_ref, o_ref, *, stacked):
    hb = h_ref[0] if stacked else h_ref[...]
    s = lax.rsqrt(jnp.maximum(deg_ref[...], 1.0))
    y = jnp.dot(hb, w_ref[...], preferred_element_type=jnp.float32) * s
    for c in range(NCHUNK):
        o_ref[c] = y[:, c * CW:(c + 1) * CW]


def _ymm(h, w, deg, src_idx=None):
    stacked = h.ndim == 3
    if stacked:
        h_spec = pl.BlockSpec((1, BN, H), lambda i: (src_idx, i, 0))
    else:
        h_spec = pl.BlockSpec((BN, H), lambda i: (i, 0))
    return pl.pallas_call(
        functools.partial(_ymm_body, stacked=stacked),
        grid=(GRID,),
        in_specs=[
            h_spec,
            pl.BlockSpec((H, H), lambda i: (0, 0)),
            pl.BlockSpec((BN, 1), lambda i: (i, 0)),
        ],
        out_specs=pl.BlockSpec((NCHUNK, BN, CW), lambda i: (0, i, 0)),
        out_shape=jax.ShapeDtypeStruct((NCHUNK, N, CW), jnp.float32),
    )(h, w, deg)


def _merge_body(a_ref, dg_ref, b_ref, o_ref):
    for dt, ens in enumerate(DST_LISTS):
        acc = None
        for en in ens:
            cat = jnp.concatenate([a_ref[en, c] for c in range(NCHUNK)], axis=-1)
            s = lax.rsqrt(jnp.maximum(dg_ref[en], 1.0))
            term = cat * s + b_ref[en][None, :]
            acc = term if acc is None else acc + term
        o_ref[dt] = jnp.maximum(acc, 0.0)


def _merge(agg, deg_in, biases):
    return pl.pallas_call(
        _merge_body,
        grid=(GRID,),
        in_specs=[
            pl.BlockSpec((6, NCHUNK, BN, CW), lambda i: (0, 0, i, 0)),
            pl.BlockSpec((6, BN, 1), lambda i: (0, i, 0)),
            pl.BlockSpec((6, H), lambda i: (0, 0)),
        ],
        out_specs=pl.BlockSpec((4, BN, H), lambda i: (0, i, 0)),
        out_shape=jax.ShapeDtypeStruct((4, N, H), jnp.float32),
    )(agg, deg_in, biases)


_MESH = plsc.VectorSubcoreMesh(core_axis_name="c", subcore_axis_name="s")


def _deg_body(edges_ref, out_ref, acc, ibuf, ones_v, zbuf, wbuf):
    cid = lax.axis_index("c")
    sid = lax.axis_index("s")
    zero16 = jnp.zeros((16,), jnp.float32)
    one16 = jnp.ones((16,), jnp.float32)

    def init(k, _):
        zbuf[pl.ds(k * 16, 16)] = zero16
        return _
    lax.fori_loop(0, SPAN // 16 + 1, init, None)

    def init_ones(k, _):
        ones_v[pl.ds(k * 16, 16)] = one16
        return _
    lax.fori_loop(0, EB // 16, init_ones, None)

    for k in range(6):
        ti = cid * 6 + k
        en = ti // 2
        row = ti - 2 * en
        ebase = (en * 2 + row) * EP

        pltpu.sync_copy(zbuf.at[pl.ds(0, SPAN)], acc.at[pl.ds(sid * SPAN, SPAN)])
        plsc.subcore_barrier()

        def eb(bi, _):
            e0 = ebase + (sid * NBLK + bi) * EB
            pltpu.sync_copy(edges_ref.at[pl.ds(e0, EB)], ibuf)
            pltpu.sync_copy(ones_v, acc.at[ibuf], add=True)
            return _
        lax.fori_loop(0, NBLK, eb, None)

        plsc.subcore_barrier()

        @pl.when(sid < 15)
        def _w():
            pltpu.sync_copy(acc.at[pl.ds(sid * SPAN, SPAN)], wbuf.at[pl.ds(0, SPAN)])
            pltpu.sync_copy(wbuf.at[pl.ds(0, SPAN)],
                            out_ref.at[pl.ds(ti * N + sid * SPAN, SPAN)])

        @pl.when(sid == 15)
        def _w2():
            pltpu.sync_copy(acc.at[pl.ds(15 * SPAN, WSPAN_LAST)], wbuf.at[pl.ds(0, WSPAN_LAST)])
            pltpu.sync_copy(wbuf.at[pl.ds(0, WSPAN_LAST)],
                            out_ref.at[pl.ds(ti * N + 15 * SPAN, WSPAN_LAST)])

        plsc.subcore_barrier()


def _deg(edges_flat):
    k = pl.kernel(
        _deg_body,
        out_type=jax.ShapeDtypeStruct((12 * N,), jnp.float32),
        mesh=_MESH,
        scratch_types=[
            pltpu.VMEM_SHARED((NJ,), jnp.float32),
            pltpu.VMEM((EB,), jnp.int32),
            pltpu.VMEM((EB,), jnp.float32),
            pltpu.VMEM((SPAN + 16,), jnp.float32),
            pltpu.VMEM((SPAN,), jnp.float32),
        ],
        compiler_params=pltpu.CompilerParams(use_tc_tiling_on_sc=False),
    )
    return k(edges_flat).reshape(12, N)


_PIECE = 256


def _scat_body(y0, y1, y2, y3, y4, y5, edges_ref, out_ref,
               acc, sbuf, dbuf, sadj, rows, zwbuf, sem):
    ytabs = [y0, y1, y2, y3, y4, y5]
    cid = lax.axis_index("c")
    sid = lax.axis_index("s")
    zero16 = jnp.zeros((16,), jnp.float32)

    def zrow(r, _):
        zwbuf[r, pl.ds(0, 16)] = zero16
        zwbuf[r, pl.ds(16, 16)] = zero16
        return _

    for j in range(2):
        c = cid + 2 * j
        off = c * N
        for en in range(6):
            ytab = ytabs[en]
            ebase = en * 2 * EP
            lax.fori_loop(0, _PIECE, zrow, None)
            for kk in range(12):
                pltpu.sync_copy(zwbuf, acc.at[pl.ds(sid * SPAN + kk * _PIECE, _PIECE)])
            pltpu.sync_copy(zwbuf.at[pl.ds(0, 56)],
                            acc.at[pl.ds(sid * SPAN + 12 * _PIECE, 56)])
            plsc.subcore_barrier()

            def eb(t, _):
                e0 = ebase + (sid * NBLK + t) * EB
                pltpu.sync_copy(edges_ref.at[pl.ds(e0, EB)], sbuf)
                pltpu.sync_copy(edges_ref.at[pl.ds(e0 + EP, EB)], dbuf)

                def adj(jj, _a):
                    v = sbuf[pl.ds(jj * 16, 16)]
                    sadj[pl.ds(jj * 16, 16)] = jnp.where(v < N, v, 0) + off
                    return _a
                lax.fori_loop(0, EB // 16, adj, None)
                pltpu.async_copy(ytab.at[sadj], rows, sem).wait()
                pltpu.sync_copy(rows, acc.at[dbuf], add=True)
                return _
            lax.fori_loop(0, NBLK, eb, None)

            plsc.subcore_barrier()

            for kk in range(12):
                po = kk * _PIECE
                pltpu.sync_copy(acc.at[pl.ds(sid * SPAN + po, _PIECE)], zwbuf)
                pltpu.sync_copy(zwbuf,
                                out_ref.at[en, c, pl.ds(sid * SPAN + po, _PIECE)])

            @pl.when(sid < 15)
            def _w():
                pltpu.sync_copy(acc.at[pl.ds(sid * SPAN + 3072, 56)],
                                zwbuf.at[pl.ds(0, 56)])
                pltpu.sync_copy(zwbuf.at[pl.ds(0, 56)],
                                out_ref.at[en, c, pl.ds(sid * SPAN + 3072, 56)])

            @pl.when(sid == 15)
            def _w2():
                pltpu.sync_copy(acc.at[pl.ds(15 * SPAN + 3072, 8)],
                                zwbuf.at[pl.ds(0, 8)])
                pltpu.sync_copy(zwbuf.at[pl.ds(0, 8)],
                                out_ref.at[en, c, pl.ds(15 * SPAN + 3072, 8)])

            plsc.subcore_barrier()


def _scat(ys, edges_all):
    k = pl.kernel(
        _scat_body,
        out_type=jax.ShapeDtypeStruct((6, NCHUNK, N, CW), jnp.float32),
        mesh=_MESH,
        scratch_types=[
            pltpu.VMEM_SHARED((NJ, CW), jnp.float32),
            pltpu.VMEM((EB,), jnp.int32),
            pltpu.VMEM((EB,), jnp.int32),
            pltpu.VMEM((EB,), jnp.int32),
            pltpu.VMEM((EB, CW), jnp.float32),
            pltpu.VMEM((_PIECE, CW), jnp.float32),
            pltpu.SemaphoreType.DMA,
        ],
        compiler_params=pltpu.CompilerParams(use_tc_tiling_on_sc=False),
    )
    flat = [y.reshape(NCHUNK * N, CW) for y in ys]
    return k(*flat, edges_all)


def kernel(feat_Policy, Win_Policy, bin_Policy, feat_Control, Win_Control, bin_Control, feat_ComplianceRequirement, Win_ComplianceRequirement, bin_ComplianceRequirement, feat_Risk, Win_Risk, bin_Risk, edge_governs, l1_W_governs, l1_b_governs, l2_W_governs, l2_b_governs, edge_governed_by, l1_W_governed_by, l1_b_governed_by, l2_W_governed_by, l2_b_governed_by, edge_requires, l1_W_requires, l1_b_requires, l2_W_requires, l2_b_requires, edge_satisfies, l1_W_satisfies, l1_b_satisfies, l2_W_satisfies, l2_b_satisfies, edge_mitigates, l1_W_mitigates, l1_b_mitigates, l2_W_mitigates, l2_b_mitigates, edge_mitigated_by, l1_W_mitigated_by, l1_b_mitigated_by, l2_W_mitigated_by, l2_b_mitigated_by):
    d = dict(locals())

    edges_all = jnp.stack([
        jnp.pad(d[f"edge_{en}"], ((0, 0), (0, EP - E)), constant_values=N)
        for en in EN_NAMES
    ]).reshape(6 * 2 * EP)

    deg_all = _deg(edges_all)
    deg_out = [deg_all[2 * en].reshape(N, 1) for en in range(6)]
    deg_in = deg_all[1::2][:, :, None]

    feats = ["Policy", "Control", "ComplianceRequirement", "Risk"]
    h = jnp.stack([_proj(d[f"feat_{nt}"], d[f"Win_{nt}"], d[f"bin_{nt}"])
                   for nt in feats])

    for lp in ("l1", "l2"):
        biases = jnp.stack([d[f"{lp}_b_{en}"] for en in EN_NAMES])
        ys = [_ymm(h, d[f"{lp}_W_{EN_NAMES[en]}"], deg_out[en], SRC_OF_EN[en])
              for en in range(6)]
        agg = _scat(ys, edges_all)
        h = _merge(agg, deg_in, biases)

    return h


# speedup vs baseline: 6.0813x; 1.7932x over previous
import functools

import jax
import jax.numpy as jnp
from jax import lax
from jax.experimental import pallas as pl
from jax.experimental.pallas import tpu as pltpu
from jax.experimental.pallas import tpu_sc as plsc

N = 50000
H = 128
E = 625000
F_IN = 385
CW = 32
NCHUNK = 4
EB = 128
NBLK = 306
EP = 16 * NBLK * EB
NBT = EP // EB
NJ = N + 48
SPAN = NJ // 16
WSPAN_LAST = N - 15 * SPAN
BN = 1000
GRID = N // BN

EN_NAMES = ["governs", "governed_by", "requires", "satisfies", "mitigates", "mitigated_by"]
SRC_OF_EN = [0, 1, 2, 1, 1, 3]
DST_LISTS = [[1], [0, 2, 5], [3], [4]]


def _proj_body(x_ref, w_ref, b_ref, o_ref):
    y = jnp.dot(x_ref[...], w_ref[...], preferred_element_type=jnp.float32)
    o_ref[...] = jnp.maximum(y + b_ref[...], 0.0)


def _proj(x, w, b):
    return pl.pallas_call(
        _proj_body,
        grid=(GRID,),
        in_specs=[
            pl.BlockSpec((BN, F_IN), lambda i: (i, 0)),
            pl.BlockSpec((F_IN, H), lambda i: (0, 0)),
            pl.BlockSpec((1, H), lambda i: (0, 0)),
        ],
        out_specs=pl.BlockSpec((BN, H), lambda i: (i, 0)),
        out_shape=jax.ShapeDtypeStruct((N, H), jnp.float32),
    )(x, w, b.reshape(1, H))


def _ymm_body(h_ref, w_ref, deg_ref, o_ref, *, stacked):
    hb = h_ref[0] if stacked else h_ref[...]
    s = lax.rsqrt(jnp.maximum(deg_ref[...], 1.0))
    y = jnp.dot(hb, w_ref[...], preferred_element_type=jnp.float32) * s
    for c in range(NCHUNK):
        o_ref[c] = y[:, c * CW:(c + 1) * CW]


def _ymm(h, w, deg, src_idx=None):
    stacked = h.ndim == 3
    if stacked:
        h_spec = pl.BlockSpec((1, BN, H), lambda i: (src_idx, i, 0))
    else:
        h_spec = pl.BlockSpec((BN, H), lambda i: (i, 0))
    return pl.pallas_call(
        functools.partial(_ymm_body, stacked=stacked),
        grid=(GRID,),
        in_specs=[
            h_spec,
            pl.BlockSpec((H, H), lambda i: (0, 0)),
            pl.BlockSpec((BN, 1), lambda i: (i, 0)),
        ],
        out_specs=pl.BlockSpec((NCHUNK, BN, CW), lambda i: (0, i, 0)),
        out_shape=jax.ShapeDtypeStruct((NCHUNK, N, CW), jnp.float32),
    )(h, w, deg)


def _merge_body(a_ref, dg_ref, b_ref, o_ref):
    for dt, ens in enumerate(DST_LISTS):
        acc = None
        for en in ens:
            cat = jnp.concatenate([a_ref[en, c] for c in range(NCHUNK)], axis=-1)
            s = lax.rsqrt(jnp.maximum(dg_ref[en], 1.0))
            term = cat * s + b_ref[en][None, :]
            acc = term if acc is None else acc + term
        o_ref[dt] = jnp.maximum(acc, 0.0)


def _merge(agg, deg_in, biases):
    return pl.pallas_call(
        _merge_body,
        grid=(GRID,),
        in_specs=[
            pl.BlockSpec((6, NCHUNK, BN, CW), lambda i: (0, 0, i, 0)),
            pl.BlockSpec((6, BN, 1), lambda i: (0, i, 0)),
            pl.BlockSpec((6, H), lambda i: (0, 0)),
        ],
        out_specs=pl.BlockSpec((4, BN, H), lambda i: (0, i, 0)),
        out_shape=jax.ShapeDtypeStruct((4, N, H), jnp.float32),
    )(agg, deg_in, biases)


_MESH = plsc.VectorSubcoreMesh(core_axis_name="c", subcore_axis_name="s")


def _deg_body(edges_ref, out_ref, acc, ibuf, ones_v, zbuf, wbuf):
    cid = lax.axis_index("c")
    sid = lax.axis_index("s")
    zero16 = jnp.zeros((16,), jnp.float32)
    one16 = jnp.ones((16,), jnp.float32)

    def init(k, _):
        zbuf[pl.ds(k * 16, 16)] = zero16
        return _
    lax.fori_loop(0, SPAN // 16 + 1, init, None)

    def init_ones(k, _):
        ones_v[0, pl.ds(k * 16, 16)] = one16
        return _
    lax.fori_loop(0, EB // 16, init_ones, None)

    for k in range(6):
        ti = cid * 6 + k
        en = ti // 2
        row = ti - 2 * en

        pltpu.sync_copy(zbuf.at[pl.ds(0, SPAN)], acc.at[pl.ds(sid * SPAN, SPAN)])
        plsc.subcore_barrier()

        def eb(bi, _):
            r0 = (en * NBT + sid * NBLK + bi) * 2 + row
            pltpu.sync_copy(edges_ref.at[pl.ds(r0, 1)], ibuf)
            pltpu.sync_copy(ones_v.at[0], acc.at[ibuf.at[0]], add=True)
            return _
        lax.fori_loop(0, NBLK, eb, None)

        plsc.subcore_barrier()

        @pl.when(sid < 15)
        def _w():
            pltpu.sync_copy(acc.at[pl.ds(sid * SPAN, SPAN)], wbuf.at[pl.ds(0, SPAN)])
            pltpu.sync_copy(wbuf.at[pl.ds(0, SPAN)],
                            out_ref.at[pl.ds(ti * N + sid * SPAN, SPAN)])

        @pl.when(sid == 15)
        def _w2():
            pltpu.sync_copy(acc.at[pl.ds(15 * SPAN, WSPAN_LAST)], wbuf.at[pl.ds(0, WSPAN_LAST)])
            pltpu.sync_copy(wbuf.at[pl.ds(0, WSPAN_LAST)],
                            out_ref.at[pl.ds(ti * N + 15 * SPAN, WSPAN_LAST)])

        plsc.subcore_barrier()


def _deg(edges_flat):
    k = pl.kernel(
        _deg_body,
        out_type=jax.ShapeDtypeStruct((12 * N,), jnp.float32),
        mesh=_MESH,
        scratch_types=[
            pltpu.VMEM_SHARED((NJ,), jnp.float32),
            pltpu.VMEM((1, EB), jnp.int32),
            pltpu.VMEM((1, EB), jnp.float32),
            pltpu.VMEM((SPAN + 16,), jnp.float32),
            pltpu.VMEM((SPAN,), jnp.float32),
        ],
        compiler_params=pltpu.CompilerParams(use_tc_tiling_on_sc=False),
    )
    return k(edges_flat).reshape(12, N)


_PIECE = 256


def _scat_body(y0, y1, y2, y3, y4, y5, edges_ref, out_ref,
               acc, ib0, ib1, sa0, sa1, rw0, rw1, zwbuf, sem0, sem1):
    ytabs = [y0, y1, y2, y3, y4, y5]
    cid = lax.axis_index("c")
    sid = lax.axis_index("s")
    zero16 = jnp.zeros((16,), jnp.float32)
    slots = ((ib0, sa0, rw0, sem0), (ib1, sa1, rw1, sem1))

    def zrow(r, _):
        zwbuf[r, pl.ds(0, 16)] = zero16
        zwbuf[r, pl.ds(16, 16)] = zero16
        return _

    for j in range(2):
        c = cid + 2 * j
        off = c * N
        for en in range(6):
            ytab = ytabs[en]
            bbase = en * NBT + sid * NBLK
            lax.fori_loop(0, _PIECE, zrow, None)
            for kk in range(12):
                pltpu.sync_copy(zwbuf, acc.at[pl.ds(sid * SPAN + kk * _PIECE, _PIECE)])
            pltpu.sync_copy(zwbuf.at[pl.ds(0, 56)],
                            acc.at[pl.ds(sid * SPAN + 12 * _PIECE, 56)])
            plsc.subcore_barrier()

            def fetch(t, ib, sa, rw, sem):
                pltpu.sync_copy(edges_ref.at[pl.ds((bbase + t) * 2, 2)], ib)

                def adj(jj, _a):
                    v = ib[0, pl.ds(jj * 16, 16)]
                    sa[pl.ds(jj * 16, 16)] = jnp.where(v < N, v, 0) + off
                    return _a
                lax.fori_loop(0, EB // 16, adj, None)
                pltpu.async_copy(ytab.at[sa], rw, sem)

            for sl in (0, 1):
                fetch(sl, *slots[sl])

            def eb(tt, _):
                for sl in (0, 1):
                    ib, sa, rw, sem = slots[sl]
                    t = tt * 2 + sl
                    pltpu.make_async_copy(ytab.at[sa], rw, sem).wait()
                    pltpu.sync_copy(rw, acc.at[ib.at[1]], add=True)

                    @pl.when(t + 2 < NBLK)
                    def _nxt():
                        fetch(t + 2, ib, sa, rw, sem)
                return _
            lax.fori_loop(0, NBLK // 2, eb, None)

            plsc.subcore_barrier()

            for kk in range(12):
                po = kk * _PIECE
                pltpu.sync_copy(acc.at[pl.ds(sid * SPAN + po, _PIECE)], zwbuf)
                pltpu.sync_copy(zwbuf,
                                out_ref.at[en, c, pl.ds(sid * SPAN + po, _PIECE)])

            @pl.when(sid < 15)
            def _w():
                pltpu.sync_copy(acc.at[pl.ds(sid * SPAN + 3072, 56)],
                                zwbuf.at[pl.ds(0, 56)])
                pltpu.sync_copy(zwbuf.at[pl.ds(0, 56)],
                                out_ref.at[en, c, pl.ds(sid * SPAN + 3072, 56)])

            @pl.when(sid == 15)
            def _w2():
                pltpu.sync_copy(acc.at[pl.ds(15 * SPAN + 3072, 8)],
                                zwbuf.at[pl.ds(0, 8)])
                pltpu.sync_copy(zwbuf.at[pl.ds(0, 8)],
                                out_ref.at[en, c, pl.ds(15 * SPAN + 3072, 8)])

            plsc.subcore_barrier()


def _scat(ys, edges_all):
    k = pl.kernel(
        _scat_body,
        out_type=jax.ShapeDtypeStruct((6, NCHUNK, N, CW), jnp.float32),
        mesh=_MESH,
        scratch_types=[
            pltpu.VMEM_SHARED((NJ, CW), jnp.float32),
            pltpu.VMEM((2, EB), jnp.int32),
            pltpu.VMEM((2, EB), jnp.int32),
            pltpu.VMEM((EB,), jnp.int32),
            pltpu.VMEM((EB,), jnp.int32),
            pltpu.VMEM((EB, CW), jnp.float32),
            pltpu.VMEM((EB, CW), jnp.float32),
            pltpu.VMEM((_PIECE, CW), jnp.float32),
            pltpu.SemaphoreType.DMA,
            pltpu.SemaphoreType.DMA,
        ],
        compiler_params=pltpu.CompilerParams(use_tc_tiling_on_sc=False),
    )
    flat = [y.reshape(NCHUNK * N, CW) for y in ys]
    return k(*flat, edges_all)


def kernel(feat_Policy, Win_Policy, bin_Policy, feat_Control, Win_Control, bin_Control, feat_ComplianceRequirement, Win_ComplianceRequirement, bin_ComplianceRequirement, feat_Risk, Win_Risk, bin_Risk, edge_governs, l1_W_governs, l1_b_governs, l2_W_governs, l2_b_governs, edge_governed_by, l1_W_governed_by, l1_b_governed_by, l2_W_governed_by, l2_b_governed_by, edge_requires, l1_W_requires, l1_b_requires, l2_W_requires, l2_b_requires, edge_satisfies, l1_W_satisfies, l1_b_satisfies, l2_W_satisfies, l2_b_satisfies, edge_mitigates, l1_W_mitigates, l1_b_mitigates, l2_W_mitigates, l2_b_mitigates, edge_mitigated_by, l1_W_mitigated_by, l1_b_mitigated_by, l2_W_mitigated_by, l2_b_mitigated_by):
    d = dict(locals())

    edges_all = jnp.stack([
        jnp.pad(d[f"edge_{en}"], ((0, 0), (0, EP - E)), constant_values=N)
        .reshape(2, NBT, EB).transpose(1, 0, 2)
        for en in EN_NAMES
    ]).reshape(6 * NBT * 2, EB)

    deg_all = _deg(edges_all)
    deg_out = [deg_all[2 * en].reshape(N, 1) for en in range(6)]
    deg_in = deg_all[1::2][:, :, None]

    feats = ["Policy", "Control", "ComplianceRequirement", "Risk"]
    h = jnp.stack([_proj(d[f"feat_{nt}"], d[f"Win_{nt}"], d[f"bin_{nt}"])
                   for nt in feats])

    for lp in ("l1", "l2"):
        biases = jnp.stack([d[f"{lp}_b_{en}"] for en in EN_NAMES])
        ys = [_ymm(h, d[f"{lp}_W_{EN_NAMES[en]}"], deg_out[en], SRC_OF_EN[en])
              for en in range(6)]
        agg = _scat(ys, edges_all)
        h = _merge(agg, deg_in, biases)

    return h


# speedup vs baseline: 7.5355x; 1.2391x over previous
import functools

import jax
import jax.numpy as jnp
from jax import lax
from jax.experimental import pallas as pl
from jax.experimental.pallas import tpu as pltpu
from jax.experimental.pallas import tpu_sc as plsc

N = 50000
H = 128
E = 625000
F_IN = 385
CW = 32
NCHUNK = 4
EB = 128
NBLK = 306
EP = 16 * NBLK * EB
NBT = EP // EB
NJ = N + 48
SPAN = NJ // 16
WSPAN_LAST = N - 15 * SPAN
BN = 1000
GRID = N // BN

EN_NAMES = ["governs", "governed_by", "requires", "satisfies", "mitigates", "mitigated_by"]
SRC_OF_EN = [0, 1, 2, 1, 1, 3]
DST_LISTS = [[1], [0, 2, 5], [3], [4]]


def _proj_body(x_ref, w_ref, b_ref, o_ref):
    y = jnp.dot(x_ref[...], w_ref[...], preferred_element_type=jnp.float32)
    o_ref[...] = jnp.maximum(y + b_ref[...], 0.0)


def _proj(x, w, b):
    return pl.pallas_call(
        _proj_body,
        grid=(GRID,),
        in_specs=[
            pl.BlockSpec((BN, F_IN), lambda i: (i, 0)),
            pl.BlockSpec((F_IN, H), lambda i: (0, 0)),
            pl.BlockSpec((1, H), lambda i: (0, 0)),
        ],
        out_specs=pl.BlockSpec((BN, H), lambda i: (i, 0)),
        out_shape=jax.ShapeDtypeStruct((N, H), jnp.float32),
    )(x, w, b.reshape(1, H))


def _ymm_body(h_ref, w_ref, deg_ref, o_ref, *, stacked):
    hb = h_ref[0] if stacked else h_ref[...]
    s = lax.rsqrt(jnp.maximum(deg_ref[...], 1.0))
    o_ref[...] = jnp.dot(hb, w_ref[...], preferred_element_type=jnp.float32) * s


def _ymm(h, w, deg, src_idx=None):
    stacked = h.ndim == 3
    if stacked:
        h_spec = pl.BlockSpec((1, BN, H), lambda i: (src_idx, i, 0))
    else:
        h_spec = pl.BlockSpec((BN, H), lambda i: (i, 0))
    return pl.pallas_call(
        functools.partial(_ymm_body, stacked=stacked),
        grid=(GRID,),
        in_specs=[
            h_spec,
            pl.BlockSpec((H, H), lambda i: (0, 0)),
            pl.BlockSpec((BN, 1), lambda i: (i, 0)),
        ],
        out_specs=pl.BlockSpec((BN, H), lambda i: (i, 0)),
        out_shape=jax.ShapeDtypeStruct((N, H), jnp.float32),
    )(h, w, deg)


def _merge_body(a_ref, dg_ref, b_ref, o_ref):
    for dt, ens in enumerate(DST_LISTS):
        acc = None
        for en in ens:
            s = lax.rsqrt(jnp.maximum(dg_ref[en], 1.0))
            term = a_ref[en] * s + b_ref[en][None, :]
            acc = term if acc is None else acc + term
        o_ref[dt] = jnp.maximum(acc, 0.0)


def _merge(agg, deg_in, biases):
    return pl.pallas_call(
        _merge_body,
        grid=(GRID,),
        in_specs=[
            pl.BlockSpec((6, BN, H), lambda i: (0, i, 0)),
            pl.BlockSpec((6, BN, 1), lambda i: (0, i, 0)),
            pl.BlockSpec((6, H), lambda i: (0, 0)),
        ],
        out_specs=pl.BlockSpec((4, BN, H), lambda i: (0, i, 0)),
        out_shape=jax.ShapeDtypeStruct((4, N, H), jnp.float32),
    )(agg, deg_in, biases)


_MESH = plsc.VectorSubcoreMesh(core_axis_name="c", subcore_axis_name="s")


def _deg_body(edges_ref, out_ref, acc, ibuf, ones_v, zbuf, wbuf):
    cid = lax.axis_index("c")
    sid = lax.axis_index("s")
    zero16 = jnp.zeros((16,), jnp.float32)
    one16 = jnp.ones((16,), jnp.float32)

    def init(k, _):
        zbuf[pl.ds(k * 16, 16)] = zero16
        return _
    lax.fori_loop(0, SPAN // 16 + 1, init, None)

    def init_ones(k, _):
        ones_v[0, pl.ds(k * 16, 16)] = one16
        return _
    lax.fori_loop(0, EB // 16, init_ones, None)

    for k in range(6):
        ti = cid * 6 + k
        en = ti // 2
        row = ti - 2 * en

        pltpu.sync_copy(zbuf.at[pl.ds(0, SPAN)], acc.at[pl.ds(sid * SPAN, SPAN)])
        plsc.subcore_barrier()

        def eb(bi, _):
            r0 = (en * NBT + sid * NBLK + bi) * 2 + row
            pltpu.sync_copy(edges_ref.at[pl.ds(r0, 1)], ibuf)
            pltpu.sync_copy(ones_v.at[0], acc.at[ibuf.at[0]], add=True)
            return _
        lax.fori_loop(0, NBLK, eb, None)

        plsc.subcore_barrier()

        @pl.when(sid < 15)
        def _w():
            pltpu.sync_copy(acc.at[pl.ds(sid * SPAN, SPAN)], wbuf.at[pl.ds(0, SPAN)])
            pltpu.sync_copy(wbuf.at[pl.ds(0, SPAN)],
                            out_ref.at[pl.ds(ti * N + sid * SPAN, SPAN)])

        @pl.when(sid == 15)
        def _w2():
            pltpu.sync_copy(acc.at[pl.ds(15 * SPAN, WSPAN_LAST)], wbuf.at[pl.ds(0, WSPAN_LAST)])
            pltpu.sync_copy(wbuf.at[pl.ds(0, WSPAN_LAST)],
                            out_ref.at[pl.ds(ti * N + 15 * SPAN, WSPAN_LAST)])

        plsc.subcore_barrier()


def _deg(edges_flat):
    k = pl.kernel(
        _deg_body,
        out_type=jax.ShapeDtypeStruct((12 * N,), jnp.float32),
        mesh=_MESH,
        scratch_types=[
            pltpu.VMEM_SHARED((NJ,), jnp.float32),
            pltpu.VMEM((1, EB), jnp.int32),
            pltpu.VMEM((1, EB), jnp.float32),
            pltpu.VMEM((SPAN + 16,), jnp.float32),
            pltpu.VMEM((SPAN,), jnp.float32),
        ],
        compiler_params=pltpu.CompilerParams(use_tc_tiling_on_sc=False),
    )
    return k(edges_flat).reshape(12, N)


_PIECE = 256


def _scat_body(y0, y1, y2, y3, y4, y5, edges_ref, out_ref,
               acc, ib0, ib1, sa0, sa1, rw0, rw1, zwbuf, sem0, sem1):
    ytabs = [y0, y1, y2, y3, y4, y5]
    cid = lax.axis_index("c")
    sid = lax.axis_index("s")
    zero16 = jnp.zeros((16,), jnp.float32)
    slots = ((ib0, sa0, rw0, sem0), (ib1, sa1, rw1, sem1))

    def zrow(r, _):
        zwbuf[r, pl.ds(0, 16)] = zero16
        zwbuf[r, pl.ds(16, 16)] = zero16
        return _

    for j in range(2):
        c = cid + 2 * j
        for en in range(6):
            ytab = ytabs[en]
            bbase = en * NBT + sid * NBLK
            lax.fori_loop(0, _PIECE, zrow, None)
            for kk in range(12):
                pltpu.sync_copy(zwbuf, acc.at[pl.ds(sid * SPAN + kk * _PIECE, _PIECE)])
            pltpu.sync_copy(zwbuf.at[pl.ds(0, 56)],
                            acc.at[pl.ds(sid * SPAN + 12 * _PIECE, 56)])
            plsc.subcore_barrier()

            def fetch(t, ib, sa, rw, sem):
                pltpu.sync_copy(edges_ref.at[pl.ds((bbase + t) * 2, 2)], ib)

                def adj(jj, _a):
                    v = ib[0, pl.ds(jj * 16, 16)]
                    sa[pl.ds(jj * 16, 16)] = jnp.where(v < N, v, 0) * NCHUNK + c
                    return _a
                lax.fori_loop(0, EB // 16, adj, None)
                pltpu.async_copy(ytab.at[sa], rw, sem)

            for sl in (0, 1):
                fetch(sl, *slots[sl])

            def eb(tt, _):
                for sl in (0, 1):
                    ib, sa, rw, sem = slots[sl]
                    t = tt * 2 + sl
                    pltpu.make_async_copy(ytab.at[sa], rw, sem).wait()
                    pltpu.sync_copy(rw, acc.at[ib.at[1]], add=True)

                    @pl.when(t + 2 < NBLK)
                    def _nxt():
                        fetch(t + 2, ib, sa, rw, sem)
                return _
            lax.fori_loop(0, NBLK // 2, eb, None)

            plsc.subcore_barrier()

            cs = c * CW
            for kk in range(12):
                po = kk * _PIECE
                pltpu.sync_copy(acc.at[pl.ds(sid * SPAN + po, _PIECE)], zwbuf)
                pltpu.sync_copy(zwbuf,
                                out_ref.at[en, pl.ds(sid * SPAN + po, _PIECE),
                                           pl.ds(cs, CW)])

            @pl.when(sid < 15)
            def _w():
                pltpu.sync_copy(acc.at[pl.ds(sid * SPAN + 3072, 56)],
                                zwbuf.at[pl.ds(0, 56)])
                pltpu.sync_copy(zwbuf.at[pl.ds(0, 56)],
                                out_ref.at[en, pl.ds(sid * SPAN + 3072, 56),
                                           pl.ds(cs, CW)])

            @pl.when(sid == 15)
            def _w2():
                pltpu.sync_copy(acc.at[pl.ds(15 * SPAN + 3072, 8)],
                                zwbuf.at[pl.ds(0, 8)])
                pltpu.sync_copy(zwbuf.at[pl.ds(0, 8)],
                                out_ref.at[en, pl.ds(15 * SPAN + 3072, 8),
                                           pl.ds(cs, CW)])

            plsc.subcore_barrier()


def _scat(ys, edges_all):
    k = pl.kernel(
        _scat_body,
        out_type=jax.ShapeDtypeStruct((6, N, H), jnp.float32),
        mesh=_MESH,
        scratch_types=[
            pltpu.VMEM_SHARED((NJ, CW), jnp.float32),
            pltpu.VMEM((2, EB), jnp.int32),
            pltpu.VMEM((2, EB), jnp.int32),
            pltpu.VMEM((EB,), jnp.int32),
            pltpu.VMEM((EB,), jnp.int32),
            pltpu.VMEM((EB, CW), jnp.float32),
            pltpu.VMEM((EB, CW), jnp.float32),
            pltpu.VMEM((_PIECE, CW), jnp.float32),
            pltpu.SemaphoreType.DMA,
            pltpu.SemaphoreType.DMA,
        ],
        compiler_params=pltpu.CompilerParams(use_tc_tiling_on_sc=False),
    )
    flat = [y.reshape(NCHUNK * N, CW) for y in ys]
    return k(*flat, edges_all)


def kernel(feat_Policy, Win_Policy, bin_Policy, feat_Control, Win_Control, bin_Control, feat_ComplianceRequirement, Win_ComplianceRequirement, bin_ComplianceRequirement, feat_Risk, Win_Risk, bin_Risk, edge_governs, l1_W_governs, l1_b_governs, l2_W_governs, l2_b_governs, edge_governed_by, l1_W_governed_by, l1_b_governed_by, l2_W_governed_by, l2_b_governed_by, edge_requires, l1_W_requires, l1_b_requires, l2_W_requires, l2_b_requires, edge_satisfies, l1_W_satisfies, l1_b_satisfies, l2_W_satisfies, l2_b_satisfies, edge_mitigates, l1_W_mitigates, l1_b_mitigates, l2_W_mitigates, l2_b_mitigates, edge_mitigated_by, l1_W_mitigated_by, l1_b_mitigated_by, l2_W_mitigated_by, l2_b_mitigated_by):
    d = dict(locals())

    edges_all = jnp.stack([
        jnp.pad(d[f"edge_{en}"], ((0, 0), (0, EP - E)), constant_values=N)
        .reshape(2, NBT, EB).transpose(1, 0, 2)
        for en in EN_NAMES
    ]).reshape(6 * NBT * 2, EB)

    deg_all = _deg(edges_all)
    deg_out = [deg_all[2 * en].reshape(N, 1) for en in range(6)]
    deg_in = deg_all[1::2][:, :, None]

    feats = ["Policy", "Control", "ComplianceRequirement", "Risk"]
    h = jnp.stack([_proj(d[f"feat_{nt}"], d[f"Win_{nt}"], d[f"bin_{nt}"])
                   for nt in feats])

    for lp in ("l1", "l2"):
        biases = jnp.stack([d[f"{lp}_b_{en}"] for en in EN_NAMES])
        ys = [_ymm(h, d[f"{lp}_W_{EN_NAMES[en]}"], deg_out[en], SRC_OF_EN[en])
              for en in range(6)]
        agg = _scat(ys, edges_all)
        h = _merge(agg, deg_in, biases)

    return h


# speedup vs baseline: 8.3436x; 1.1072x over previous
import functools

import jax
import jax.numpy as jnp
from jax import lax
from jax.experimental import pallas as pl
from jax.experimental.pallas import tpu as pltpu
from jax.experimental.pallas import tpu_sc as plsc

N = 50000
H = 128
E = 625000
F_IN = 385
CW = 32
NCHUNK = 4
EB = 128
NBLK = 306
EP = 16 * NBLK * EB
NBT = EP // EB
NJ = N + 48
SPAN = NJ // 16
WSPAN_LAST = N - 15 * SPAN
BN = 1000
GRID = N // BN

EN_NAMES = ["governs", "governed_by", "requires", "satisfies", "mitigates", "mitigated_by"]
SRC_OF_EN = [0, 1, 2, 1, 1, 3]
DST_LISTS = [[1], [0, 2, 5], [3], [4]]


def _proj_body(x_ref, w_ref, b_ref, o_ref):
    y = jnp.dot(x_ref[...], w_ref[...], preferred_element_type=jnp.float32)
    o_ref[...] = jnp.maximum(y + b_ref[...], 0.0)


def _proj(x, w, b):
    return pl.pallas_call(
        _proj_body,
        grid=(GRID,),
        in_specs=[
            pl.BlockSpec((BN, F_IN), lambda i: (i, 0)),
            pl.BlockSpec((F_IN, H), lambda i: (0, 0)),
            pl.BlockSpec((1, H), lambda i: (0, 0)),
        ],
        out_specs=pl.BlockSpec((BN, H), lambda i: (i, 0)),
        out_shape=jax.ShapeDtypeStruct((N, H), jnp.float32),
    )(x, w, b.reshape(1, H))


def _ymm_body(h_ref, w_ref, deg_ref, o_ref, *, stacked):
    hb = h_ref[0] if stacked else h_ref[...]
    s = lax.rsqrt(jnp.maximum(deg_ref[...], 1.0))
    o_ref[...] = jnp.dot(hb, w_ref[...], preferred_element_type=jnp.float32) * s


def _ymm(h, w, deg, src_idx=None):
    stacked = h.ndim == 3
    if stacked:
        h_spec = pl.BlockSpec((1, BN, H), lambda i: (src_idx, i, 0))
    else:
        h_spec = pl.BlockSpec((BN, H), lambda i: (i, 0))
    return pl.pallas_call(
        functools.partial(_ymm_body, stacked=stacked),
        grid=(GRID,),
        in_specs=[
            h_spec,
            pl.BlockSpec((H, H), lambda i: (0, 0)),
            pl.BlockSpec((BN, 1), lambda i: (i, 0)),
        ],
        out_specs=pl.BlockSpec((BN, H), lambda i: (i, 0)),
        out_shape=jax.ShapeDtypeStruct((N, H), jnp.float32),
    )(h, w, deg)


def _merge_body(a_ref, dg_ref, b_ref, o_ref):
    for dt, ens in enumerate(DST_LISTS):
        acc = None
        for en in ens:
            s = lax.rsqrt(jnp.maximum(dg_ref[en], 1.0))
            term = a_ref[en] * s + b_ref[en][None, :]
            acc = term if acc is None else acc + term
        o_ref[dt] = jnp.maximum(acc, 0.0)


def _merge(agg, deg_in, biases):
    return pl.pallas_call(
        _merge_body,
        grid=(GRID,),
        in_specs=[
            pl.BlockSpec((6, BN, H), lambda i: (0, i, 0)),
            pl.BlockSpec((6, BN, 1), lambda i: (0, i, 0)),
            pl.BlockSpec((6, H), lambda i: (0, 0)),
        ],
        out_specs=pl.BlockSpec((4, BN, H), lambda i: (0, i, 0)),
        out_shape=jax.ShapeDtypeStruct((4, N, H), jnp.float32),
    )(agg, deg_in, biases)


_MESH = plsc.VectorSubcoreMesh(core_axis_name="c", subcore_axis_name="s")


def _deg_body(edges_ref, out_ref, acc, ib0, ib1, ones_v, zbuf, wbuf, sm0, sm1):
    cid = lax.axis_index("c")
    sid = lax.axis_index("s")
    zero16 = jnp.zeros((16,), jnp.float32)
    one16 = jnp.ones((16,), jnp.float32)

    def init(k, _):
        zbuf[pl.ds(k * 16, 16)] = zero16
        return _
    lax.fori_loop(0, SPAN // 16 + 1, init, None)

    def init_ones(k, _):
        ones_v[0, pl.ds(k * 16, 16)] = one16
        return _
    lax.fori_loop(0, EB // 16, init_ones, None)

    for k in range(6):
        ti = cid * 6 + k
        en = ti // 2
        row = ti - 2 * en

        pltpu.sync_copy(zbuf.at[pl.ds(0, SPAN)], acc.at[pl.ds(sid * SPAN, SPAN)])
        plsc.subcore_barrier()

        rbase = (en * NBT + sid * NBLK) * 2 + row
        dslots = ((ib0, sm0), (ib1, sm1))

        def dfetch(bi, ib, sm):
            pltpu.async_copy(edges_ref.at[pl.ds(rbase + bi * 2, 1)], ib, sm)

        for sl in (0, 1):
            dfetch(sl, *dslots[sl])

        def eb(tt, _):
            for sl in (0, 1):
                ib, sm = dslots[sl]
                bi = tt * 2 + sl
                pltpu.make_async_copy(edges_ref.at[pl.ds(rbase + bi * 2, 1)],
                                      ib, sm).wait()
                pltpu.sync_copy(ones_v.at[0], acc.at[ib.at[0]], add=True)

                @pl.when(bi + 2 < NBLK)
                def _nx():
                    dfetch(bi + 2, ib, sm)
            return _
        lax.fori_loop(0, NBLK // 2, eb, None)

        plsc.subcore_barrier()

        @pl.when(sid < 15)
        def _w():
            pltpu.sync_copy(acc.at[pl.ds(sid * SPAN, SPAN)], wbuf.at[pl.ds(0, SPAN)])
            pltpu.sync_copy(wbuf.at[pl.ds(0, SPAN)],
                            out_ref.at[pl.ds(ti * N + sid * SPAN, SPAN)])

        @pl.when(sid == 15)
        def _w2():
            pltpu.sync_copy(acc.at[pl.ds(15 * SPAN, WSPAN_LAST)], wbuf.at[pl.ds(0, WSPAN_LAST)])
            pltpu.sync_copy(wbuf.at[pl.ds(0, WSPAN_LAST)],
                            out_ref.at[pl.ds(ti * N + 15 * SPAN, WSPAN_LAST)])

        plsc.subcore_barrier()


def _deg(edges_flat):
    k = pl.kernel(
        _deg_body,
        out_type=jax.ShapeDtypeStruct((12 * N,), jnp.float32),
        mesh=_MESH,
        scratch_types=[
            pltpu.VMEM_SHARED((NJ,), jnp.float32),
            pltpu.VMEM((1, EB), jnp.int32),
            pltpu.VMEM((1, EB), jnp.int32),
            pltpu.VMEM((1, EB), jnp.float32),
            pltpu.VMEM((SPAN + 16,), jnp.float32),
            pltpu.VMEM((SPAN,), jnp.float32),
            pltpu.SemaphoreType.DMA,
            pltpu.SemaphoreType.DMA,
        ],
        compiler_params=pltpu.CompilerParams(use_tc_tiling_on_sc=False),
    )
    return k(edges_flat).reshape(12, N)


_PIECE = 256


def _scat_body(y0, y1, y2, y3, y4, y5, edges_ref, out_ref,
               acc, ib0, ib1, ib2, sa0, sa1, sa2, rw0, rw1, rw2,
               zwbuf, sem0, sem1, sem2):
    ytabs = [y0, y1, y2, y3, y4, y5]
    cid = lax.axis_index("c")
    sid = lax.axis_index("s")
    zero16 = jnp.zeros((16,), jnp.float32)
    slots = ((ib0, sa0, rw0, sem0), (ib1, sa1, rw1, sem1), (ib2, sa2, rw2, sem2))
    nsl = len(slots)

    def zrow(r, _):
        zwbuf[r, pl.ds(0, 16)] = zero16
        zwbuf[r, pl.ds(16, 16)] = zero16
        return _

    for j in range(2):
        c = cid + 2 * j
        for en in range(6):
            ytab = ytabs[en]
            bbase = en * NBT + sid * NBLK
            lax.fori_loop(0, _PIECE, zrow, None)
            for kk in range(12):
                pltpu.sync_copy(zwbuf, acc.at[pl.ds(sid * SPAN + kk * _PIECE, _PIECE)])
            pltpu.sync_copy(zwbuf.at[pl.ds(0, 56)],
                            acc.at[pl.ds(sid * SPAN + 12 * _PIECE, 56)])
            plsc.subcore_barrier()

            def fetch(t, ib, sa, rw, sem):
                pltpu.sync_copy(edges_ref.at[pl.ds((bbase + t) * 2, 2)], ib)

                def adj(jj, _a):
                    v = ib[0, pl.ds(jj * 16, 16)]
                    sa[pl.ds(jj * 16, 16)] = jnp.where(v < N, v, 0) * NCHUNK + c
                    return _a
                lax.fori_loop(0, EB // 16, adj, None)
                pltpu.async_copy(ytab.at[sa], rw, sem)

            for sl in range(nsl):
                fetch(sl, *slots[sl])

            def eb(tt, _):
                for sl in range(nsl):
                    ib, sa, rw, sem = slots[sl]
                    t = tt * nsl + sl
                    pltpu.make_async_copy(ytab.at[sa], rw, sem).wait()
                    pltpu.sync_copy(rw, acc.at[ib.at[1]], add=True)

                    @pl.when(t + nsl < NBLK)
                    def _nxt():
                        fetch(t + nsl, ib, sa, rw, sem)
                return _
            lax.fori_loop(0, NBLK // nsl, eb, None)

            plsc.subcore_barrier()

            cs = c * CW
            for kk in range(12):
                po = kk * _PIECE
                pltpu.sync_copy(acc.at[pl.ds(sid * SPAN + po, _PIECE)], zwbuf)
                pltpu.sync_copy(zwbuf,
                                out_ref.at[en, pl.ds(sid * SPAN + po, _PIECE),
                                           pl.ds(cs, CW)])

            @pl.when(sid < 15)
            def _w():
                pltpu.sync_copy(acc.at[pl.ds(sid * SPAN + 3072, 56)],
                                zwbuf.at[pl.ds(0, 56)])
                pltpu.sync_copy(zwbuf.at[pl.ds(0, 56)],
                                out_ref.at[en, pl.ds(sid * SPAN + 3072, 56),
                                           pl.ds(cs, CW)])

            @pl.when(sid == 15)
            def _w2():
                pltpu.sync_copy(acc.at[pl.ds(15 * SPAN + 3072, 8)],
                                zwbuf.at[pl.ds(0, 8)])
                pltpu.sync_copy(zwbuf.at[pl.ds(0, 8)],
                                out_ref.at[en, pl.ds(15 * SPAN + 3072, 8),
                                           pl.ds(cs, CW)])

            plsc.subcore_barrier()


def _scat(ys, edges_all):
    k = pl.kernel(
        _scat_body,
        out_type=jax.ShapeDtypeStruct((6, N, H), jnp.float32),
        mesh=_MESH,
        scratch_types=[
            pltpu.VMEM_SHARED((NJ, CW), jnp.float32),
            pltpu.VMEM((2, EB), jnp.int32),
            pltpu.VMEM((2, EB), jnp.int32),
            pltpu.VMEM((2, EB), jnp.int32),
            pltpu.VMEM((EB,), jnp.int32),
            pltpu.VMEM((EB,), jnp.int32),
            pltpu.VMEM((EB,), jnp.int32),
            pltpu.VMEM((EB, CW), jnp.float32),
            pltpu.VMEM((EB, CW), jnp.float32),
            pltpu.VMEM((EB, CW), jnp.float32),
            pltpu.VMEM((_PIECE, CW), jnp.float32),
            pltpu.SemaphoreType.DMA,
            pltpu.SemaphoreType.DMA,
            pltpu.SemaphoreType.DMA,
        ],
        compiler_params=pltpu.CompilerParams(use_tc_tiling_on_sc=False),
    )
    flat = [y.reshape(NCHUNK * N, CW) for y in ys]
    return k(*flat, edges_all)


def kernel(feat_Policy, Win_Policy, bin_Policy, feat_Control, Win_Control, bin_Control, feat_ComplianceRequirement, Win_ComplianceRequirement, bin_ComplianceRequirement, feat_Risk, Win_Risk, bin_Risk, edge_governs, l1_W_governs, l1_b_governs, l2_W_governs, l2_b_governs, edge_governed_by, l1_W_governed_by, l1_b_governed_by, l2_W_governed_by, l2_b_governed_by, edge_requires, l1_W_requires, l1_b_requires, l2_W_requires, l2_b_requires, edge_satisfies, l1_W_satisfies, l1_b_satisfies, l2_W_satisfies, l2_b_satisfies, edge_mitigates, l1_W_mitigates, l1_b_mitigates, l2_W_mitigates, l2_b_mitigates, edge_mitigated_by, l1_W_mitigated_by, l1_b_mitigated_by, l2_W_mitigated_by, l2_b_mitigated_by):
    d = dict(locals())

    edges_all = jnp.stack([
        jnp.pad(d[f"edge_{en}"], ((0, 0), (0, EP - E)), constant_values=N)
        .reshape(2, NBT, EB).transpose(1, 0, 2)
        for en in EN_NAMES
    ]).reshape(6 * NBT * 2, EB)

    deg_all = _deg(edges_all)
    deg_out = [deg_all[2 * en].reshape(N, 1) for en in range(6)]
    deg_in = deg_all[1::2][:, :, None]

    feats = ["Policy", "Control", "ComplianceRequirement", "Risk"]
    h = jnp.stack([_proj(d[f"feat_{nt}"], d[f"Win_{nt}"], d[f"bin_{nt}"])
                   for nt in feats])

    for lp in ("l1", "l2"):
        biases = jnp.stack([d[f"{lp}_b_{en}"] for en in EN_NAMES])
        ys = [_ymm(h, d[f"{lp}_W_{EN_NAMES[en]}"], deg_out[en], SRC_OF_EN[en])
              for en in range(6)]
        agg = _scat(ys, edges_all)
        h = _merge(agg, deg_in, biases)

    return h


# speedup vs baseline: 12.0652x; 1.4460x over previous
import functools

import jax
import jax.numpy as jnp
from jax import lax
from jax.experimental import pallas as pl
from jax.experimental.pallas import tpu as pltpu
from jax.experimental.pallas import tpu_sc as plsc

N = 50000
H = 128
E = 625000
F_IN = 385
CW = 32
NCHUNK = 4
EB = 128
NBLK = 306
EP = 16 * NBLK * EB
NBT = EP // EB
NJ = N + 48
SPAN = NJ // 16
WSPAN_LAST = N - 15 * SPAN
BN = 1000
GRID = N // BN

EN_NAMES = ["governs", "governed_by", "requires", "satisfies", "mitigates", "mitigated_by"]
SRC_OF_EN = [0, 1, 2, 1, 1, 3]
DST_LISTS = [[1], [0, 2, 5], [3], [4]]


def _proj_body(x_ref, w_ref, b_ref, o_ref):
    y = jnp.dot(x_ref[...], w_ref[...], preferred_element_type=jnp.float32)
    o_ref[...] = jnp.maximum(y + b_ref[...], 0.0)


def _proj(x, w, b):
    return pl.pallas_call(
        _proj_body,
        grid=(GRID,),
        in_specs=[
            pl.BlockSpec((BN, F_IN), lambda i: (i, 0)),
            pl.BlockSpec((F_IN, H), lambda i: (0, 0)),
            pl.BlockSpec((1, H), lambda i: (0, 0)),
        ],
        out_specs=pl.BlockSpec((BN, H), lambda i: (i, 0)),
        out_shape=jax.ShapeDtypeStruct((N, H), jnp.float32),
    )(x, w, b.reshape(1, H))


def _ymm_body(h_ref, w_ref, deg_ref, o_ref, *, stacked):
    hb = h_ref[0] if stacked else h_ref[...]
    s = lax.rsqrt(jnp.maximum(deg_ref[...], 1.0))
    o_ref[...] = jnp.dot(hb, w_ref[...], preferred_element_type=jnp.float32) * s


def _ymm(h, w, deg, src_idx=None):
    stacked = h.ndim == 3
    if stacked:
        h_spec = pl.BlockSpec((1, BN, H), lambda i: (src_idx, i, 0))
    else:
        h_spec = pl.BlockSpec((BN, H), lambda i: (i, 0))
    return pl.pallas_call(
        functools.partial(_ymm_body, stacked=stacked),
        grid=(GRID,),
        in_specs=[
            h_spec,
            pl.BlockSpec((H, H), lambda i: (0, 0)),
            pl.BlockSpec((BN, 1), lambda i: (i, 0)),
        ],
        out_specs=pl.BlockSpec((BN, H), lambda i: (i, 0)),
        out_shape=jax.ShapeDtypeStruct((N, H), jnp.float32),
    )(h, w, deg)


def _merge_body(a_ref, dg_ref, b_ref, o_ref):
    for dt, ens in enumerate(DST_LISTS):
        acc = None
        for en in ens:
            s = lax.rsqrt(jnp.maximum(dg_ref[en], 1.0))
            term = a_ref[en] * s + b_ref[en][None, :]
            acc = term if acc is None else acc + term
        o_ref[dt] = jnp.maximum(acc, 0.0)


def _merge(agg, deg_in, biases):
    return pl.pallas_call(
        _merge_body,
        grid=(GRID,),
        in_specs=[
            pl.BlockSpec((6, BN, H), lambda i: (0, i, 0)),
            pl.BlockSpec((6, BN, 1), lambda i: (0, i, 0)),
            pl.BlockSpec((6, H), lambda i: (0, 0)),
        ],
        out_specs=pl.BlockSpec((4, BN, H), lambda i: (0, i, 0)),
        out_shape=jax.ShapeDtypeStruct((4, N, H), jnp.float32),
    )(agg, deg_in, biases)


_MESH = plsc.VectorSubcoreMesh(core_axis_name="c", subcore_axis_name="s")


def _deg_body(edges_ref, out_ref, acc, ib0, ib1, ones_v, zbuf, wbuf, sm0, sm1):
    cid = lax.axis_index("c")
    sid = lax.axis_index("s")
    zero16 = jnp.zeros((16,), jnp.float32)
    one16 = jnp.ones((16,), jnp.float32)

    def init(k, _):
        zbuf[pl.ds(k * 16, 16)] = zero16
        return _
    lax.fori_loop(0, SPAN // 16 + 1, init, None)

    def init_ones(k, _):
        ones_v[0, pl.ds(k * 16, 16)] = one16
        return _
    lax.fori_loop(0, EB // 16, init_ones, None)

    for k in range(6):
        ti = cid * 6 + k
        en = ti // 2
        row = ti - 2 * en

        pltpu.sync_copy(zbuf.at[pl.ds(0, SPAN)], acc.at[pl.ds(sid * SPAN, SPAN)])
        plsc.subcore_barrier()

        rbase = (en * NBT + sid * NBLK) * 2 + row
        dslots = ((ib0, sm0), (ib1, sm1))

        def dfetch(bi, ib, sm):
            pltpu.async_copy(edges_ref.at[pl.ds(rbase + bi * 2, 1)], ib, sm)

        for sl in (0, 1):
            dfetch(sl, *dslots[sl])

        def eb(tt, _):
            for sl in (0, 1):
                ib, sm = dslots[sl]
                bi = tt * 2 + sl
                pltpu.make_async_copy(edges_ref.at[pl.ds(rbase + bi * 2, 1)],
                                      ib, sm).wait()
                pltpu.sync_copy(ones_v.at[0], acc.at[ib.at[0]], add=True)

                @pl.when(bi + 2 < NBLK)
                def _nx():
                    dfetch(bi + 2, ib, sm)
            return _
        lax.fori_loop(0, NBLK // 2, eb, None)

        plsc.subcore_barrier()

        @pl.when(sid < 15)
        def _w():
            pltpu.sync_copy(acc.at[pl.ds(sid * SPAN, SPAN)], wbuf.at[pl.ds(0, SPAN)])
            pltpu.sync_copy(wbuf.at[pl.ds(0, SPAN)],
                            out_ref.at[pl.ds(ti * N + sid * SPAN, SPAN)])

        @pl.when(sid == 15)
        def _w2():
            pltpu.sync_copy(acc.at[pl.ds(15 * SPAN, WSPAN_LAST)], wbuf.at[pl.ds(0, WSPAN_LAST)])
            pltpu.sync_copy(wbuf.at[pl.ds(0, WSPAN_LAST)],
                            out_ref.at[pl.ds(ti * N + 15 * SPAN, WSPAN_LAST)])

        plsc.subcore_barrier()


def _deg(edges_flat):
    k = pl.kernel(
        _deg_body,
        out_type=jax.ShapeDtypeStruct((12 * N,), jnp.float32),
        mesh=_MESH,
        scratch_types=[
            pltpu.VMEM_SHARED((NJ,), jnp.float32),
            pltpu.VMEM((1, EB), jnp.int32),
            pltpu.VMEM((1, EB), jnp.int32),
            pltpu.VMEM((1, EB), jnp.float32),
            pltpu.VMEM((SPAN + 16,), jnp.float32),
            pltpu.VMEM((SPAN,), jnp.float32),
            pltpu.SemaphoreType.DMA,
            pltpu.SemaphoreType.DMA,
        ],
        compiler_params=pltpu.CompilerParams(use_tc_tiling_on_sc=False),
    )
    return k(edges_flat).reshape(12, N)


_PIECE = 256


def _scat_body(y0, y1, y2, y3, y4, y5, edges_ref, out_ref,
               acc, ib0, ib1, ib2, sa0, sa1, sa2, db0, db1, db2,
               rw0, rw1, rw2, zwbuf, sem0, sem1, sem2, sei0, sei1, sei2):
    ytabs = [y0, y1, y2, y3, y4, y5]
    cid = lax.axis_index("c")
    sid = lax.axis_index("s")
    zero16 = jnp.zeros((16,), jnp.float32)
    slots = ((ib0, sa0, db0, rw0, sem0, sei0), (ib1, sa1, db1, rw1, sem1, sei1),
             (ib2, sa2, db2, rw2, sem2, sei2))
    nsl = len(slots)

    def zrow(r, _):
        zwbuf[r, pl.ds(0, 16)] = zero16
        zwbuf[r, pl.ds(16, 16)] = zero16
        return _

    for j in range(2):
        c = cid + 2 * j
        for en in range(6):
            ytab = ytabs[en]
            bbase = en * NBT + sid * NBLK
            lax.fori_loop(0, _PIECE, zrow, None)
            for kk in range(12):
                pltpu.sync_copy(zwbuf, acc.at[pl.ds(sid * SPAN + kk * _PIECE, _PIECE)])
            pltpu.sync_copy(zwbuf.at[pl.ds(0, 56)],
                            acc.at[pl.ds(sid * SPAN + 12 * _PIECE, 56)])
            plsc.subcore_barrier()

            def fetch_idx(t, ib, sei):
                pltpu.async_copy(edges_ref.at[pl.ds((bbase + t) * 2, 2)], ib, sei)

            def launch_gather(t, ib, sa, db, rw, sem, sei):
                pltpu.make_async_copy(edges_ref.at[pl.ds((bbase + t) * 2, 2)],
                                      ib, sei).wait()

                def adj(jj, _a):
                    v = ib[0, pl.ds(jj * 16, 16)]
                    sa[pl.ds(jj * 16, 16)] = jnp.where(v < N, v, 0) * NCHUNK + c
                    db[0, pl.ds(jj * 16, 16)] = ib[1, pl.ds(jj * 16, 16)]
                    return _a
                lax.fori_loop(0, EB // 16, adj, None)
                pltpu.async_copy(ytab.at[sa], rw, sem)

            for sl in range(nsl):
                fetch_idx(sl, slots[sl][0], slots[sl][5])
            for sl in range(nsl):
                ib, sa, db, rw, sem, sei = slots[sl]
                launch_gather(sl, ib, sa, db, rw, sem, sei)
                fetch_idx(sl + nsl, ib, sei)

            def eb(tt, _):
                for sl in range(nsl):
                    ib, sa, db, rw, sem, sei = slots[sl]
                    t = tt * nsl + sl
                    pltpu.make_async_copy(ytab.at[sa], rw, sem).wait()
                    pltpu.sync_copy(rw, acc.at[db.at[0]], add=True)

                    @pl.when(t + nsl < NBLK)
                    def _nxt():
                        launch_gather(t + nsl, ib, sa, db, rw, sem, sei)

                    @pl.when(t + 2 * nsl < NBLK)
                    def _pf():
                        fetch_idx(t + 2 * nsl, ib, sei)
                return _
            lax.fori_loop(0, NBLK // nsl, eb, None)

            plsc.subcore_barrier()

            cs = c * CW
            for kk in range(12):
                po = kk * _PIECE
                pltpu.sync_copy(acc.at[pl.ds(sid * SPAN + po, _PIECE)], zwbuf)
                pltpu.sync_copy(zwbuf,
                                out_ref.at[en, pl.ds(sid * SPAN + po, _PIECE),
                                           pl.ds(cs, CW)])

            @pl.when(sid < 15)
            def _w():
                pltpu.sync_copy(acc.at[pl.ds(sid * SPAN + 3072, 56)],
                                zwbuf.at[pl.ds(0, 56)])
                pltpu.sync_copy(zwbuf.at[pl.ds(0, 56)],
                                out_ref.at[en, pl.ds(sid * SPAN + 3072, 56),
                                           pl.ds(cs, CW)])

            @pl.when(sid == 15)
            def _w2():
                pltpu.sync_copy(acc.at[pl.ds(15 * SPAN + 3072, 8)],
                                zwbuf.at[pl.ds(0, 8)])
                pltpu.sync_copy(zwbuf.at[pl.ds(0, 8)],
                                out_ref.at[en, pl.ds(15 * SPAN + 3072, 8),
                                           pl.ds(cs, CW)])

            plsc.subcore_barrier()


def _scat(ys, edges_all):
    k = pl.kernel(
        _scat_body,
        out_type=jax.ShapeDtypeStruct((6, N, H), jnp.float32),
        mesh=_MESH,
        scratch_types=[
            pltpu.VMEM_SHARED((NJ, CW), jnp.float32),
            pltpu.VMEM((2, EB), jnp.int32),
            pltpu.VMEM((2, EB), jnp.int32),
            pltpu.VMEM((2, EB), jnp.int32),
            pltpu.VMEM((EB,), jnp.int32),
            pltpu.VMEM((EB,), jnp.int32),
            pltpu.VMEM((EB,), jnp.int32),
            pltpu.VMEM((1, EB), jnp.int32),
            pltpu.VMEM((1, EB), jnp.int32),
            pltpu.VMEM((1, EB), jnp.int32),
            pltpu.VMEM((EB, CW), jnp.float32),
            pltpu.VMEM((EB, CW), jnp.float32),
            pltpu.VMEM((EB, CW), jnp.float32),
            pltpu.VMEM((_PIECE, CW), jnp.float32),
            pltpu.SemaphoreType.DMA,
            pltpu.SemaphoreType.DMA,
            pltpu.SemaphoreType.DMA,
            pltpu.SemaphoreType.DMA,
            pltpu.SemaphoreType.DMA,
            pltpu.SemaphoreType.DMA,
        ],
        compiler_params=pltpu.CompilerParams(use_tc_tiling_on_sc=False),
    )
    flat = [y.reshape(NCHUNK * N, CW) for y in ys]
    return k(*flat, edges_all)


def kernel(feat_Policy, Win_Policy, bin_Policy, feat_Control, Win_Control, bin_Control, feat_ComplianceRequirement, Win_ComplianceRequirement, bin_ComplianceRequirement, feat_Risk, Win_Risk, bin_Risk, edge_governs, l1_W_governs, l1_b_governs, l2_W_governs, l2_b_governs, edge_governed_by, l1_W_governed_by, l1_b_governed_by, l2_W_governed_by, l2_b_governed_by, edge_requires, l1_W_requires, l1_b_requires, l2_W_requires, l2_b_requires, edge_satisfies, l1_W_satisfies, l1_b_satisfies, l2_W_satisfies, l2_b_satisfies, edge_mitigates, l1_W_mitigates, l1_b_mitigates, l2_W_mitigates, l2_b_mitigates, edge_mitigated_by, l1_W_mitigated_by, l1_b_mitigated_by, l2_W_mitigated_by, l2_b_mitigated_by):
    d = dict(locals())

    edges_all = jnp.stack([
        jnp.pad(d[f"edge_{en}"], ((0, 0), (0, EP - E)), constant_values=N)
        .reshape(2, NBT, EB).transpose(1, 0, 2)
        for en in EN_NAMES
    ]).reshape(6 * NBT * 2, EB)

    deg_all = _deg(edges_all)
    deg_out = [deg_all[2 * en].reshape(N, 1) for en in range(6)]
    deg_in = deg_all[1::2][:, :, None]

    feats = ["Policy", "Control", "ComplianceRequirement", "Risk"]
    h = jnp.stack([_proj(d[f"feat_{nt}"], d[f"Win_{nt}"], d[f"bin_{nt}"])
                   for nt in feats])

    for lp in ("l1", "l2"):
        biases = jnp.stack([d[f"{lp}_b_{en}"] for en in EN_NAMES])
        ys = [_ymm(h, d[f"{lp}_W_{EN_NAMES[en]}"], deg_out[en], SRC_OF_EN[en])
              for en in range(6)]
        agg = _scat(ys, edges_all)
        h = _merge(agg, deg_in, biases)

    return h
